# Initial kernel scaffold; baseline (speedup 1.0000x reference)
import jax
import jax.numpy as jnp
from jax.experimental import pallas as pl


def kernel(x, edge_index, edge_attr, batch, bn_node_g, bn_node_b, bn_edge_g, bn_edge_b, gine_lin_W, gine_lin_b, mlp_W1, mlp_b1, mlp_W2, mlp_b2, ln_g, ln_b, in_W, conv_W, conv_b, xproj_W, dt_W, dt_b, A_log, D_param, out_W, cls_ln1_g, cls_ln1_b, cls_W1, cls_b1, cls_ln2_g, cls_ln2_b, cls_W2, cls_b2, cls_ln3_g, cls_ln3_b, cls_W3, cls_b3):
    raise NotImplementedError("write your pallas kernel here")


# speedup vs baseline: 14.8040x; 14.8040x over previous
import functools

import jax
import jax.numpy as jnp
from jax import lax
from jax.experimental import pallas as pl
from jax.experimental.pallas import tpu as pltpu
from jax.experimental.pallas import tpu_sc as plsc

N = 10000
E = 320000
D = 128
DI = 256
DS = 16


_SC_MESH = plsc.VectorSubcoreMesh(core_axis_name="c", subcore_axis_name="s")
_NW = 32
_PER_W = E // _NW
_CHUNK = 80
_NCHUNK = _PER_W // _CHUNK
_ROWS_PER_TILE = N // 16


def _sc_gather_body(table_hbm, idx_hbm, out_hbm, ibuf, rbuf, sem):
    c = lax.axis_index("c")
    s = lax.axis_index("s")
    base = (s * 2 + c) * _PER_W

    def step(j, carry):
        off = base + j * _CHUNK
        pltpu.sync_copy(idx_hbm.at[pl.ds(off, _CHUNK)], ibuf)
        pltpu.async_copy(table_hbm.at[ibuf], rbuf, sem).wait()
        pltpu.sync_copy(rbuf, out_hbm.at[pl.ds(off, _CHUNK)])
        return carry

    lax.fori_loop(0, _NCHUNK, step, 0)


def _sc_gather(table, idx):
    return pl.kernel(
        _sc_gather_body,
        out_type=jax.ShapeDtypeStruct((E, D), jnp.float32),
        mesh=_SC_MESH,
        scratch_types=[
            pltpu.VMEM((_CHUNK,), jnp.int32),
            pltpu.VMEM((_CHUNK, D), jnp.float32),
            pltpu.SemaphoreType.DMA,
        ],
    )(table, idx)


_HALF = N // 2
_ACC_ROWS = 5008
_TROWS = 312
_PER_W_SC = E // 16
_NCHUNK_SC = _PER_W_SC // _CHUNK


def _sc_scatter_body(msg_hbm, dst_hbm, zeros_hbm, out_hbm, ibuf, rbuf, zbuf, acc):
    c = lax.axis_index("c")
    s = lax.axis_index("s")
    row0 = s * _TROWS
    lo = c * _HALF
    pltpu.sync_copy(zeros_hbm, zbuf)
    pltpu.sync_copy(zbuf.at[pl.ds(0, _TROWS)], acc.at[pl.ds(row0, _TROWS)])

    @pl.when(s == 0)
    def _():
        pltpu.sync_copy(zbuf.at[pl.ds(0, 16)], acc.at[pl.ds(16 * _TROWS, 16)])

    plsc.subcore_barrier()

    base = s * _PER_W_SC

    def step(j, carry):
        off = base + j * _CHUNK
        pltpu.sync_copy(dst_hbm.at[pl.ds(off, _CHUNK)], ibuf)
        pltpu.sync_copy(msg_hbm.at[pl.ds(off, _CHUNK)], rbuf)
        for i in range(_CHUNK // 16):
            v = ibuf[pl.ds(i * 16, 16)] - lo
            inr = (v >= 0) & (v < _HALF)
            ibuf[pl.ds(i * 16, 16)] = jnp.where(inr, v, _HALF)
        pltpu.sync_copy(rbuf, acc.at[ibuf], add=True)
        return carry

    lax.fori_loop(0, _NCHUNK_SC, step, 0)
    plsc.subcore_barrier()
    pltpu.sync_copy(acc.at[pl.ds(row0, _TROWS)],
                    out_hbm.at[c, pl.ds(row0, _TROWS)])

    @pl.when(s == 0)
    def _():
        pltpu.sync_copy(acc.at[pl.ds(16 * _TROWS, 16)],
                        out_hbm.at[c, pl.ds(16 * _TROWS, 16)])


def _sc_scatter_add(msg, dst, zeros312):
    return pl.kernel(
        _sc_scatter_body,
        out_type=jax.ShapeDtypeStruct((2, _ACC_ROWS, D), jnp.float32),
        mesh=_SC_MESH,
        scratch_types=[
            pltpu.VMEM((_CHUNK,), jnp.int32),
            pltpu.VMEM((_CHUNK, D), jnp.float32),
            pltpu.VMEM((_TROWS, D), jnp.float32),
            pltpu.VMEM_SHARED((_ACC_ROWS, D), jnp.float32),
        ],
    )(msg, dst, zeros312)


def _bn_kernel(x_ref, g_ref, b_ref, o_ref):
    xv = x_ref[...]
    m = jnp.mean(xv, axis=0, keepdims=True)
    v = jnp.mean((xv - m) ** 2, axis=0, keepdims=True)
    o_ref[...] = (xv - m) * lax.rsqrt(v + 1e-5) * g_ref[...] + b_ref[...]


def _tc_batch_norm_x(x, g, b):
    return pl.pallas_call(
        _bn_kernel,
        out_shape=jax.ShapeDtypeStruct((N, D), jnp.float32),
    )(x, g.reshape(1, D), b.reshape(1, D))


_EA_R = 2000
_EA_G = E // _EA_R


def _ea_stats_kernel(ea_ref, o_ref):
    @pl.when(pl.program_id(0) == 0)
    def _():
        o_ref[...] = jnp.zeros_like(o_ref)

    ev = ea_ref[...]
    o_ref[0:1, :] += jnp.sum(ev, axis=0, keepdims=True)
    o_ref[1:2, :] += jnp.sum(ev * ev, axis=0, keepdims=True)


def _tc_ea_stats(ea):
    return pl.pallas_call(
        _ea_stats_kernel,
        grid=(_EA_G,),
        in_specs=[pl.BlockSpec((_EA_R, D), lambda i: (i, 0))],
        out_specs=pl.BlockSpec((8, D), lambda i: (0, 0)),
        out_shape=jax.ShapeDtypeStruct((8, D), jnp.float32),
    )(ea)


def _edge_msg_kernel(ea_ref, gx_ref, w_ref, b_ref, o_ref):
    e = jnp.dot(ea_ref[...], w_ref[...], preferred_element_type=jnp.float32)
    o_ref[...] = jnp.maximum(e + b_ref[...] + gx_ref[...], 0.0)


def _tc_edge_msg(ea, gx, w, b):
    return pl.pallas_call(
        _edge_msg_kernel,
        grid=(_EA_G,),
        in_specs=[
            pl.BlockSpec((_EA_R, D), lambda i: (i, 0)),
            pl.BlockSpec((_EA_R, D), lambda i: (i, 0)),
            pl.BlockSpec((D, D), lambda i: (0, 0)),
            pl.BlockSpec((1, D), lambda i: (0, 0)),
        ],
        out_specs=pl.BlockSpec((_EA_R, D), lambda i: (i, 0)),
        out_shape=jax.ShapeDtypeStruct((E, D), jnp.float32),
    )(ea, gx, w, b)


def _node_mlp_kernel(xn_ref, a0_ref, w1_ref, b1_ref, w2_ref, b2_ref, o_ref):
    h = xn_ref[...] + a0_ref[...]
    hm = jnp.dot(h, w1_ref[...], preferred_element_type=jnp.float32) + b1_ref[...]
    hm = jnp.where(hm >= 0, hm, 0.01 * hm)
    h2 = jnp.dot(hm, w2_ref[...], preferred_element_type=jnp.float32) + b2_ref[...]
    o_ref[...] = jnp.maximum(h2, 0.0)


def _tc_node_mlp(xn, a0, w1, b1, w2, b2):
    return pl.pallas_call(
        _node_mlp_kernel,
        out_shape=jax.ShapeDtypeStruct((N, D), jnp.float32),
    )(xn, a0, w1, b1.reshape(1, D), w2, b2.reshape(1, D))


_MB_R = 2000
_MB_G = N // _MB_R


def _mamba_pre_kernel(h_ref, hh_ref, m3_ref, lng_ref, lnb_ref, inw_ref,
                      convt_ref, convb_ref, xproj_ref, dtp_ref, dtb_ref,
                      sel_ref, delta_ref, u_ref, gz_ref, bc_ref):
    def ln(v):
        m = jnp.mean(v, axis=1, keepdims=True)
        var = jnp.mean((v - m) ** 2, axis=1, keepdims=True)
        return (v - m) * lax.rsqrt(var + 1e-5) * lng_ref[...] + lnb_ref[...]

    hv = ln(h_ref[...])
    hh = ln(hh_ref[...])
    xz = jnp.dot(hv, inw_ref[...], preferred_element_type=jnp.float32)
    u0 = xz[:, :DI]
    z = xz[:, DI:]
    u0h = jnp.dot(hh, inw_ref[...], preferred_element_type=jnp.float32)[:, :DI]

    y = u0 * convt_ref[3:4, :]
    for k in (1, 2, 3):
        shifted = jnp.concatenate([u0h[8 - k:, :], u0[:-k, :]], axis=0)
        y = y + shifted * convt_ref[3 - k:4 - k, :] * m3_ref[:, k - 1:k]
    y = y + convb_ref[...]
    u = y * jax.nn.sigmoid(y)
    xdbl = jnp.dot(u, xproj_ref[...], preferred_element_type=jnp.float32)
    dpre = jnp.dot(xdbl, dtp_ref[...], preferred_element_type=jnp.float32)
    delta_ref[...] = jax.nn.softplus(dpre + dtb_ref[...])
    u_ref[...] = u
    gz_ref[...] = z * jax.nn.sigmoid(z)
    bc_ref[...] = jnp.dot(xdbl, sel_ref[...], preferred_element_type=jnp.float32)


def _tc_mamba_pre(h, m3, lng, lnb, inw, convt, convb, xprojp, dtp, dtb, sel):
    halo_spec = pl.BlockSpec((8, D), lambda i: (jnp.maximum(i * (_MB_R // 8) - 1, 0), 0))
    full = lambda shape: pl.BlockSpec(shape, lambda i: tuple(0 for _ in shape))
    return pl.pallas_call(
        _mamba_pre_kernel,
        grid=(_MB_G,),
        in_specs=[
            pl.BlockSpec((_MB_R, D), lambda i: (i, 0)),
            halo_spec,
            pl.BlockSpec((_MB_R, 8), lambda i: (i, 0)),
            full((1, D)), full((1, D)), full((D, 2 * DI)),
            full((8, DI)), full((1, DI)), full((DI, D)),
            full((D, DI)), full((1, DI)), full((D, 32)),
        ],
        out_specs=[
            pl.BlockSpec((_MB_R, DI), lambda i: (i, 0)),
            pl.BlockSpec((_MB_R, DI), lambda i: (i, 0)),
            pl.BlockSpec((_MB_R, DI), lambda i: (i, 0)),
            pl.BlockSpec((_MB_R, 32), lambda i: (i, 0)),
        ],
        out_shape=[
            jax.ShapeDtypeStruct((N, DI), jnp.float32),
            jax.ShapeDtypeStruct((N, DI), jnp.float32),
            jax.ShapeDtypeStruct((N, DI), jnp.float32),
            jax.ShapeDtypeStruct((N, 32), jnp.float32),
        ],
    )(h, h, m3, lng, lnb, inw, convt, convb, xprojp, dtp, dtb, sel)


_SC_T = 8
_SC_G = N // _SC_T


def _scan_kernel(delta_ref, u_ref, bct_ref, keep_ref, at_ref, ys_ref, st_ref):
    @pl.when(pl.program_id(0) == 0)
    def _():
        st_ref[...] = jnp.zeros_like(st_ref)

    st = st_ref[...]
    at = at_ref[...]
    rows = []
    for k in range(_SC_T):
        d = delta_ref[k:k + 1, :]
        dak = jnp.exp(d * at)
        duk = d * u_ref[k:k + 1, :]
        bc = bct_ref[0, :, k:k + 1]
        cc = bct_ref[0, :, _SC_T + k:_SC_T + k + 1]
        kp = keep_ref[0, 0:1, k:k + 1].reshape(1, 1)
        st = dak * (st * kp) + duk * bc
        rows.append(jnp.sum(st * cc, axis=0, keepdims=True))
    ys_ref[...] = jnp.concatenate(rows, axis=0)
    st_ref[...] = st


def _tc_scan(delta, u, bct, keep, at):
    return pl.pallas_call(
        _scan_kernel,
        grid=(_SC_G,),
        in_specs=[
            pl.BlockSpec((_SC_T, DI), lambda i: (i, 0)),
            pl.BlockSpec((_SC_T, DI), lambda i: (i, 0)),
            pl.BlockSpec((1, DS, 2 * _SC_T), lambda i: (i, 0, 0)),
            pl.BlockSpec((1, 1, _SC_T), lambda i: (i, 0, 0)),
            pl.BlockSpec((DS, DI), lambda i: (0, 0)),
        ],
        out_specs=pl.BlockSpec((_SC_T, DI), lambda i: (i, 0)),
        out_shape=jax.ShapeDtypeStruct((N, DI), jnp.float32),
        scratch_shapes=[pltpu.VMEM((DS, DI), jnp.float32)],
    )(delta, u, bct, keep, at)


def _mamba_post_kernel(ys_ref, u_ref, gz_ref, dp_ref, ow_ref, hres_ref, o_ref):
    v = (ys_ref[...] + u_ref[...] * dp_ref[...]) * gz_ref[...]
    o_ref[...] = jnp.dot(v, ow_ref[...], preferred_element_type=jnp.float32) + hres_ref[...]


def _tc_mamba_post(ys, u, gz, dp, ow, hres):
    full = lambda shape: pl.BlockSpec(shape, lambda i: tuple(0 for _ in shape))
    return pl.pallas_call(
        _mamba_post_kernel,
        grid=(_MB_G,),
        in_specs=[
            pl.BlockSpec((_MB_R, DI), lambda i: (i, 0)),
            pl.BlockSpec((_MB_R, DI), lambda i: (i, 0)),
            pl.BlockSpec((_MB_R, DI), lambda i: (i, 0)),
            full((1, DI)), full((DI, D)),
            pl.BlockSpec((_MB_R, D), lambda i: (i, 0)),
        ],
        out_specs=pl.BlockSpec((_MB_R, D), lambda i: (i, 0)),
        out_shape=jax.ShapeDtypeStruct((N, D), jnp.float32),
    )(ys, u, gz, dp, ow, hres)


_CL_R = 2000
_CL_G = E // _CL_R


def _cls_kernel(gs_ref, gd_ref, g1_ref, b1g_ref, w1_ref, b1_ref, g2_ref,
                b2g_ref, w2_ref, b2_ref, g3_ref, b3g_ref, w3_ref, b3_ref,
                o_ref):
    def ln(v, g, b):
        m = jnp.mean(v, axis=1, keepdims=True)
        var = jnp.mean((v - m) ** 2, axis=1, keepdims=True)
        return (v - m) * lax.rsqrt(var + 1e-5) * g + b

    rep = jnp.concatenate([gs_ref[...], gd_ref[...]], axis=1)
    zz = ln(rep, g1_ref[...], b1g_ref[...])
    zz = jnp.dot(zz, w1_ref[...], preferred_element_type=jnp.float32) + b1_ref[...]
    zz = jnp.where(zz >= 0, zz, 0.01 * zz)
    zz = ln(zz, g2_ref[...], b2g_ref[...])
    zz = jnp.dot(zz, w2_ref[...], preferred_element_type=jnp.float32) + b2_ref[...]
    zz = jnp.where(zz >= 0, zz, 0.01 * zz)
    zz = ln(zz, g3_ref[...], b3g_ref[...])
    zz = jnp.dot(zz, w3_ref[...], preferred_element_type=jnp.float32) + b3_ref[...]
    o_ref[...] = zz[:, :2]


def _tc_classifier(gs, gd, params):
    full = lambda shape: pl.BlockSpec(shape, lambda i: tuple(0 for _ in shape))
    in_specs = [
        pl.BlockSpec((_CL_R, D), lambda i: (i, 0)),
        pl.BlockSpec((_CL_R, D), lambda i: (i, 0)),
        full((1, 2 * D)), full((1, 2 * D)), full((2 * D, 2 * D)), full((1, 2 * D)),
        full((1, 2 * D)), full((1, 2 * D)), full((2 * D, D)), full((1, D)),
        full((1, D)), full((1, D)), full((D, D)), full((1, D)),
    ]
    return pl.pallas_call(
        _cls_kernel,
        grid=(_CL_G,),
        in_specs=in_specs,
        out_specs=pl.BlockSpec((_CL_R, 2), lambda i: (i, 0)),
        out_shape=jax.ShapeDtypeStruct((E, 2), jnp.float32),
    )(gs, gd, *params)


def kernel(x, edge_index, edge_attr, batch, bn_node_g, bn_node_b, bn_edge_g,
           bn_edge_b, gine_lin_W, gine_lin_b, mlp_W1, mlp_b1, mlp_W2, mlp_b2,
           ln_g, ln_b, in_W, conv_W, conv_b, xproj_W, dt_W, dt_b, A_log,
           D_param, out_W, cls_ln1_g, cls_ln1_b, cls_W1, cls_b1, cls_ln2_g,
           cls_ln2_b, cls_W2, cls_b2, cls_ln3_g, cls_ln3_b, cls_W3, cls_b3):
    f32 = jnp.float32
    src = edge_index[0].astype(jnp.int32)
    dst = edge_index[1].astype(jnp.int32)

    xn = _tc_batch_norm_x(x, bn_node_g, bn_node_b)
    stats = _tc_ea_stats(edge_attr)
    ea_mean = stats[0] / E
    ea_var = stats[1] / E - ea_mean * ea_mean
    scale = bn_edge_g * lax.rsqrt(ea_var + 1e-5)
    shift = bn_edge_b - ea_mean * scale
    w_fold = scale[:, None] * gine_lin_W
    b_fold = (shift @ gine_lin_W + gine_lin_b).reshape(1, D)

    gx = _sc_gather(xn, src)
    msg = _tc_edge_msg(edge_attr, gx, w_fold, b_fold)
    zeros312 = jnp.zeros((_TROWS, D), f32)
    aggr = _sc_scatter_add(msg, dst, zeros312)
    aggr_c = jnp.concatenate([aggr[0, :_HALF], aggr[1, :_HALF]], axis=0)
    h = _tc_node_mlp(xn, aggr_c, mlp_W1, mlp_b1, mlp_W2, mlp_b2)

    bprev1 = jnp.concatenate([batch[:1] - 1, batch[:-1]])
    bprev2 = jnp.concatenate([batch[:2] - 1, batch[:-2]])
    bprev3 = jnp.concatenate([batch[:3] - 1, batch[:-3]])
    m3 = jnp.zeros((N, 8), f32)
    m3 = m3.at[:, 0].set((batch == bprev1).astype(f32))
    m3 = m3.at[:, 1].set((batch == bprev2).astype(f32))
    m3 = m3.at[:, 2].set((batch == bprev3).astype(f32))
    keep = (batch == bprev1).astype(f32).reshape(_SC_G, 1, _SC_T)

    at = -jnp.exp(A_log).T
    convt = jnp.zeros((8, DI), f32).at[:4].set(conv_W.T)
    xprojp = jnp.zeros((DI, D), f32).at[:, :40].set(xproj_W)
    dtp = jnp.zeros((D, DI), f32).at[:8].set(dt_W)
    sel = jnp.zeros((D, 32), f32)
    sel = sel.at[8 + jnp.arange(16), jnp.arange(16)].set(1.0)
    sel = sel.at[24 + jnp.arange(16), 16 + jnp.arange(16)].set(1.0)
    lng = ln_g.reshape(1, D)
    lnb = ln_b.reshape(1, D)
    convb = conv_b.reshape(1, DI)
    dtb = dt_b.reshape(1, DI)
    dp = D_param.reshape(1, DI)

    def mamba_block(hcur):
        delta, u, gz, bc = _tc_mamba_pre(hcur, m3, lng, lnb, in_W, convt,
                                         convb, xprojp, dtp, dtb, sel)
        bg = bc[:, :16].reshape(_SC_G, _SC_T, DS).transpose(0, 2, 1)
        cg = bc[:, 16:32].reshape(_SC_G, _SC_T, DS).transpose(0, 2, 1)
        bct = jnp.concatenate([bg, cg], axis=2)
        ys = _tc_scan(delta, u, bct, keep, at)
        return _tc_mamba_post(ys, u, gz, dp, out_W, hcur)

    h = mamba_block(h)
    h = mamba_block(h)

    gs = _sc_gather(h, src)
    gd = _sc_gather(h, dst)
    w3p = jnp.zeros((D, D), f32).at[:, :2].set(cls_W3)
    b3p = jnp.zeros((1, D), f32).at[0, :2].set(cls_b3)
    params = (
        cls_ln1_g.reshape(1, 2 * D), cls_ln1_b.reshape(1, 2 * D), cls_W1,
        cls_b1.reshape(1, 2 * D), cls_ln2_g.reshape(1, 2 * D),
        cls_ln2_b.reshape(1, 2 * D), cls_W2, cls_b2.reshape(1, D),
        cls_ln3_g.reshape(1, D), cls_ln3_b.reshape(1, D), w3p, b3p,
    )
    return _tc_classifier(gs, gd, params)
